# Initial kernel scaffold; baseline (speedup 1.0000x reference)
import jax
import jax.numpy as jnp
from jax.experimental import pallas as pl


def kernel(h_User, h_Computer, edge_index_r1, edge_attr_r1, edge_index_r2, edge_attr_r2, Wq_r1, Wk_r1, Wv_r1, Wm_r1, bm_r1, imp_r1, Wq_r2, Wk_r2, Wv_r2, Wm_r2, bm_r2, imp_r2, Wout_User, bout_User, lng_User, lnb_User, Wout_Computer, bout_Computer, lng_Computer, lnb_Computer):
    raise NotImplementedError("write your pallas kernel here")


# speedup vs baseline: 1.1778x; 1.1778x over previous
import functools

import jax
import jax.numpy as jnp
from jax import lax
from jax.experimental import pallas as pl
from jax.experimental.pallas import tpu as pltpu

N = 10000
E = 160000
HID = 128
H = 8
DH = HID // H
SCALE = DH ** -0.5

ROW_BLK = 1000


def _dotT(x, w):
    return lax.dot_general(x, w, (((1,), (1,)), ((), ())),
                           preferred_element_type=jnp.float32)


def _qkv_body(hU, hC, wq1, wk1, wv1, wq2, wk2, wv2,
              q1, k1, v1, q2, k2, v2):
    q1[...] = _dotT(hC[...], wq1[...])
    k1[...] = _dotT(hU[...], wk1[...])
    v1[...] = _dotT(hU[...], wv1[...])
    q2[...] = _dotT(hU[...], wq2[...])
    k2[...] = _dotT(hC[...], wk2[...])
    v2[...] = _dotT(hC[...], wv2[...])


def _qkv(hU, hC, wq1, wk1, wv1, wq2, wk2, wv2):
    blk = pl.BlockSpec((ROW_BLK, HID), lambda i: (i, 0))
    wblk = pl.BlockSpec((HID, HID), lambda i: (0, 0))
    out = jax.ShapeDtypeStruct((N, HID), jnp.float32)
    return pl.pallas_call(
        _qkv_body,
        grid=(N // ROW_BLK,),
        in_specs=[blk, blk] + [wblk] * 6,
        out_specs=[blk] * 6,
        out_shape=[out] * 6,
    )(hU, hC, wq1, wk1, wv1, wq2, wk2, wv2)


def _post_body(h, agg, cnt, wm, bm, imp, wout, bout, lng, lnb, o):
    sig = 1.0 / (1.0 + jnp.exp(-imp[0, 0]))
    m = (_dotT(agg[...], wm[...]) + cnt[...] * bm[...]) * sig
    y = h[...] + _dotT(m, wout[...]) + bout[...]
    mu = jnp.mean(y, axis=-1, keepdims=True)
    var = jnp.mean((y - mu) ** 2, axis=-1, keepdims=True)
    o[...] = (y - mu) / jnp.sqrt(var + 1e-5) * lng[...] + lnb[...]


def _post(h, agg, cnt, wm, bm, imp, wout, bout, lng, lnb):
    blk = pl.BlockSpec((ROW_BLK, HID), lambda i: (i, 0))
    cblk = pl.BlockSpec((ROW_BLK, 1), lambda i: (i, 0))
    wblk = pl.BlockSpec((HID, HID), lambda i: (0, 0))
    vblk = pl.BlockSpec((1, HID), lambda i: (0, 0))
    sblk = pl.BlockSpec((1, 1), lambda i: (0, 0))
    return pl.pallas_call(
        _post_body,
        grid=(N // ROW_BLK,),
        in_specs=[blk, blk, cblk, wblk, vblk, sblk, wblk, vblk, vblk, vblk],
        out_specs=blk,
        out_shape=jax.ShapeDtypeStruct((N, HID), jnp.float32),
    )(h, agg, cnt.reshape(N, 1), wm, bm.reshape(1, HID),
      imp.reshape(1, 1), wout, bout.reshape(1, HID),
      lng.reshape(1, HID), lnb.reshape(1, HID))


def _edges(Q, K, V, src, dst, ew):
    q = Q[dst].reshape(E, H, DH)
    k = K[src].reshape(E, H, DH)
    att = jnp.sum(q * k, axis=-1) * SCALE * ew[:, None]
    ex = jnp.exp(att)
    es = jax.ops.segment_sum(ex, dst, num_segments=N)
    p = ex / (es[dst] + 1e-10)
    wv = (V[src].reshape(E, H, DH) * p[..., None]).reshape(E, HID)
    agg = jax.ops.segment_sum(wv, dst, num_segments=N)
    cnt = jax.ops.segment_sum(jnp.ones((E,), jnp.float32), dst,
                              num_segments=N)
    return agg, cnt


def kernel(h_User, h_Computer, edge_index_r1, edge_attr_r1, edge_index_r2,
           edge_attr_r2, Wq_r1, Wk_r1, Wv_r1, Wm_r1, bm_r1, imp_r1, Wq_r2,
           Wk_r2, Wv_r2, Wm_r2, bm_r2, imp_r2, Wout_User, bout_User,
           lng_User, lnb_User, Wout_Computer, bout_Computer, lng_Computer,
           lnb_Computer):
    q1, k1, v1, q2, k2, v2 = _qkv(h_User, h_Computer,
                                  Wq_r1, Wk_r1, Wv_r1, Wq_r2, Wk_r2, Wv_r2)
    agg1, cnt1 = _edges(q1, k1, v1, edge_index_r1[0], edge_index_r1[1],
                        edge_attr_r1)
    agg2, cnt2 = _edges(q2, k2, v2, edge_index_r2[0], edge_index_r2[1],
                        edge_attr_r2)
    out_User = _post(h_User, agg2, cnt2, Wm_r2, bm_r2, imp_r2,
                     Wout_User, bout_User, lng_User, lnb_User)
    out_Computer = _post(h_Computer, agg1, cnt1, Wm_r1, bm_r1, imp_r1,
                         Wout_Computer, bout_Computer, lng_Computer,
                         lnb_Computer)
    return (out_User, out_Computer)
